# Initial kernel scaffold; baseline (speedup 1.0000x reference)
import jax
import jax.numpy as jnp
from jax.experimental import pallas as pl


def kernel(x, d, grid, gate_w0, gate_w1, gate_w2, i0w0, i0w1, i0w2, i1w0, i1w1, i1w2, geo_w0, geo_w1, rgb_w0, rgb_w1, rgb_w2):
    raise NotImplementedError("write your pallas kernel here")


# speedup vs baseline: 1.0783x; 1.0783x over previous
import functools

import jax
import jax.numpy as jnp
import numpy as np
from jax.experimental import pallas as pl
from jax.experimental.pallas import tpu as pltpu

N_POINTS = 131072
L_LEVELS = 16
FDIM = 2
T_SIZE = 1 << 19
N_MIN = 16
SCALE = 0.5
B_GROWTH = float(np.exp(np.log(2048 * SCALE / N_MIN) / (L_LEVELS - 1)))
RES_LIST = [int(np.floor(N_MIN * (B_GROWTH ** l))) for l in range(L_LEVELS)]
PRIME1 = np.uint32(2654435761)
PRIME2 = np.uint32(805459861)


def _hash_encode_jax(x01, grid):
    feats = []
    for l in range(L_LEVELS):
        res = RES_LIST[l]
        pos = x01 * res
        p0 = jnp.floor(pos)
        w = pos - p0
        p0u = p0.astype(jnp.uint32)
        acc = jnp.zeros((x01.shape[0], FDIM), dtype=x01.dtype)
        for corner in range(8):
            offs = np.array([(corner >> k) & 1 for k in range(3)], dtype=np.uint32)
            c = p0u + jnp.asarray(offs)[None, :]
            idx = (c[:, 0] * np.uint32(1)) ^ (c[:, 1] * PRIME1) ^ (c[:, 2] * PRIME2)
            idx = (idx & jnp.uint32(T_SIZE - 1)).astype(jnp.int32)
            cf = jnp.take(grid[l], idx, axis=0)
            offs_f = jnp.asarray(offs.astype(np.float32))
            wt = jnp.prod(jnp.where(offs_f[None, :] > 0, w, 1.0 - w), axis=1, keepdims=True)
            acc = acc + wt * cf
        feats.append(acc)
    return jnp.concatenate(feats, axis=1)


_BLK = 4096


def _head_kernel(feat_ref, d_ref,
                 gw0, gw1, gw2, a0, a1, a2, b0, b1, b2, geo0, geo1, r0, r1, r2,
                 sig_ref, rgb_ref, gates_ref, load_ref, tidx_ref):
    i = pl.program_id(0)
    feat = feat_ref[...]

    def dot(x, w):
        return jax.lax.dot_general(x, w[...], (((1,), (0,)), ((), ())),
                                   preferred_element_type=jnp.float32)

    relu = lambda v: jnp.maximum(v, 0.0)

    g = relu(dot(relu(dot(feat, gw0)), gw1))
    logits = dot(g, gw2)
    l0 = logits[:, 0]
    l1 = logits[:, 1]
    sel = l1 > l0
    sel_f = sel.astype(jnp.float32)

    tidx_ref[...] = sel.astype(jnp.int32)[:, None]
    gates_ref[...] = jnp.stack([1.0 - sel_f, sel_f], axis=1)

    cnt1 = jnp.sum(sel_f)
    cnt = jnp.stack([jnp.float32(feat.shape[0]) - cnt1, cnt1])

    @pl.when(i == 0)
    def _():
        load_ref[...] = jnp.zeros_like(load_ref)

    load_ref[...] += cnt

    e0 = dot(relu(dot(relu(dot(feat, a0)), a1)), a2)
    e1 = dot(relu(dot(relu(dot(feat, b0)), b1)), b2)
    post = jnp.where(sel[:, None], e1, e0)

    h = dot(relu(dot(post, geo0)), geo1)
    sig_ref[...] = jnp.exp(h[:, 0])

    d = d_ref[...]
    dx = d[:, 0]
    dy = d[:, 1]
    dz = d[:, 2]
    inv = jax.lax.rsqrt(dx * dx + dy * dy + dz * dz)
    nrm = 1.0 / (1.0 / inv + 1e-8)
    x = dx * nrm
    y = dy * nrm
    z = dz * nrm
    xx = x * x
    yy = y * y
    zz = z * z
    xy = x * y
    yz = y * z
    xz = x * z
    sh_cols = [
        jnp.full_like(x, 0.28209479177387814),
        -0.48860251190291987 * y,
        0.48860251190291987 * z,
        -0.48860251190291987 * x,
        1.0925484305920792 * xy,
        -1.0925484305920792 * yz,
        0.94617469575755997 * zz - 0.31539156525251999,
        -1.0925484305920792 * xz,
        0.54627421529603959 * (xx - yy),
        0.59004358992664352 * y * (-3.0 * xx + yy),
        2.8906114426405538 * xy * z,
        0.45704579946446572 * y * (1.0 - 5.0 * zz),
        0.3731763325901154 * z * (5.0 * zz - 3.0),
        0.45704579946446572 * x * (1.0 - 5.0 * zz),
        1.4453057213202769 * z * (xx - yy),
        0.59004358992664352 * x * (-xx + 3.0 * yy),
    ]
    sh = jnp.stack(sh_cols, axis=1)
    rgb_in = jnp.concatenate([sh, h[:, 1:]], axis=1)
    r = dot(relu(dot(relu(dot(rgb_in, r0)), r1)), r2)
    rgb_ref[...] = jax.nn.sigmoid(r)


def _head(feat, d, weights):
    n = feat.shape[0]
    grid_n = n // _BLK
    row_spec = lambda width: pl.BlockSpec((_BLK, width), lambda i: (i, 0))
    full = lambda a: pl.BlockSpec(a.shape, lambda i: (0,) * a.ndim)
    out_shapes = (
        jax.ShapeDtypeStruct((n,), jnp.float32),
        jax.ShapeDtypeStruct((n, 3), jnp.float32),
        jax.ShapeDtypeStruct((n, 2), jnp.float32),
        jax.ShapeDtypeStruct((2,), jnp.float32),
        jax.ShapeDtypeStruct((n, 1), jnp.int32),
    )
    out_specs = (
        pl.BlockSpec((_BLK,), lambda i: (i,)),
        row_spec(3),
        row_spec(2),
        pl.BlockSpec((2,), lambda i: (0,)),
        row_spec(1),
    )
    return pl.pallas_call(
        _head_kernel,
        grid=(grid_n,),
        in_specs=[row_spec(32), row_spec(3)] + [full(w) for w in weights],
        out_specs=out_specs,
        out_shape=out_shapes,
    )(feat, d, *weights)


def kernel(x, d, grid, gate_w0, gate_w1, gate_w2, i0w0, i0w1, i0w2,
           i1w0, i1w1, i1w2, geo_w0, geo_w1, rgb_w0, rgb_w1, rgb_w2):
    x01 = jnp.clip((x + SCALE) / (2.0 * SCALE), 0.0, 1.0)
    feat = _hash_encode_jax(x01, grid)
    weights = (gate_w0, gate_w1, gate_w2, i0w0, i0w1, i0w2, i1w0, i1w1, i1w2,
               geo_w0, geo_w1, rgb_w0, rgb_w1, rgb_w2)
    sigmas, rgbs, gates, load, top_idx = _head(feat, d, weights)
    return (sigmas, rgbs, gates, load, top_idx)
